# Initial kernel scaffold; baseline (speedup 1.0000x reference)
import jax
import jax.numpy as jnp
from jax.experimental import pallas as pl


def kernel(points_feature, points_center, batch_idx, points_cloud_timestamp, foreground_mask, W_pf, b_pf, g_pf, be_pf, W_cl, b_cl, g_cl, be_cl):
    raise NotImplementedError("write your pallas kernel here")


# speedup vs baseline: 1.3245x; 1.3245x over previous
import jax
import jax.numpy as jnp
from jax.experimental import pallas as pl
from jax.experimental.pallas import tpu as pltpu

_D = 128
_BLK = 1024
_KSP = 500000
_KSP2 = 250000


def _leaky(x):
    return jnp.where(x > 0, x, 0.1 * x)


def _dotT(a, w):
    return jax.lax.dot_general(a, w, (((1,), (1,)), ((), ())),
                               preferred_element_type=jnp.float32)


def _x_stats_body(numc_ref, cf_ref, w_ref, b_ref, x_ref, s1_ref, s2_ref):
    i = pl.program_id(0)
    x = _dotT(cf_ref[...], w_ref[...]) + b_ref[...]
    x_ref[...] = x
    rows = i * _BLK + jax.lax.broadcasted_iota(jnp.int32, (_BLK, 1), 0)
    vf = (rows < numc_ref[0]).astype(jnp.float32)
    xm = x * vf
    s1_ref[...] = jnp.sum(xm, axis=0).reshape(1, 1, _D)
    s2_ref[...] = jnp.sum(xm * x, axis=0).reshape(1, 1, _D)


def _cfn_body(numc_ref, x_ref, m_ref, v_ref, g_ref, be_ref, o_ref):
    i = pl.program_id(0)
    xn = _leaky(g_ref[...] * (x_ref[...] - m_ref[...])
                / jnp.sqrt(v_ref[...] + 1e-5) + be_ref[...])
    rows = i * _BLK + jax.lax.broadcasted_iota(jnp.int32, (_BLK, 1), 0)
    o_ref[...] = jnp.where(rows < numc_ref[0], xn, 0.0)


def _h_stats_body(pf_ref, pfc_ref, pfa_ref, w_ref, b_ref,
                  h_ref, s1_ref, s2_ref, c_ref):
    pf, pfc, pfa = pf_ref[...], pfc_ref[...], pfa_ref[...]
    w = w_ref[...]
    h = (_dotT(pf, w[:, 0:_D]) + _dotT(pfc, w[:, _D:2 * _D])
         + _dotT(pfa, w[:, 2 * _D:3 * _D])) + b_ref[...]
    h_ref[...] = h
    mf = (jnp.sum(pfc - pfa, axis=1, keepdims=True) > 0).astype(jnp.float32)
    hm = h * mf
    s1_ref[...] = jnp.sum(hm, axis=0).reshape(1, 1, _D)
    s2_ref[...] = jnp.sum(hm * h, axis=0).reshape(1, 1, _D)
    c_ref[...] = jnp.full((1, 1, _D), jnp.sum(mf), jnp.float32)


def _out_body(cnt_ref, h_ref, pfc_ref, pfa_ref, pf_ref, m_ref, v_ref,
              g_ref, be_ref, o_ref):
    hn = _leaky(g_ref[...] * (h_ref[...] - m_ref[...])
                / jnp.sqrt(v_ref[...] + 1e-5) + be_ref[...])
    mrow = jnp.sum(pfc_ref[...] - pfa_ref[...], axis=1, keepdims=True) > 0
    sel = jnp.logical_and(mrow, cnt_ref[0] > 1.0)
    o_ref[...] = jnp.where(sel, hn, pf_ref[...])


def _row_spec():
    return pl.BlockSpec((_BLK, _D), lambda i: (i, 0))


def _full_spec(r, c):
    return pl.BlockSpec((r, c), lambda i: (0, 0))


def _smem_spec():
    return pl.BlockSpec(memory_space=pltpu.SMEM)


def _stat_spec():
    return pl.BlockSpec((1, 1, _D), lambda i: (i, 0, 0))


def _stat_shape(grid):
    return jax.ShapeDtypeStruct((grid, 1, _D), jnp.float32)


def kernel(points_feature, points_center, batch_idx, points_cloud_timestamp,
           foreground_mask, W_pf, b_pf, g_pf, be_pf, W_cl, b_cl, g_cl, be_cl):
    n = points_feature.shape[0]
    npad = ((n + _BLK - 1) // _BLK) * _BLK
    grid = npad // _BLK
    f32 = jnp.float32

    b32 = batch_idx.astype(jnp.int32)
    xi = jnp.floor((points_center[:, 0] - 0.0) / 0.02).astype(jnp.int32)
    yi = jnp.floor((points_center[:, 1] - 0.0) / 0.02).astype(jnp.int32)
    zi = jnp.floor((points_center[:, 2] - 0.0) / 0.02).astype(jnp.int32)
    key = ((b32 * 50 + xi) * 50 + yi) * 50 + zi
    present = jnp.zeros((_KSP,), jnp.int32).at[key].max(1)
    excl = jnp.cumsum(present) - present
    pci = excl[key]
    numc = jnp.max(jnp.where(foreground_mask, pci, -1)) + 1
    numc_arr = numc.reshape(1)

    fmask = foreground_mask.astype(f32)
    pfw = points_feature * fmask[:, None]
    cfs = jnp.zeros((n, _D), f32).at[pci].add(pfw)
    cfc = jnp.zeros((n,), f32).at[pci].add(fmask)
    cf = cfs / jnp.maximum(cfc, 1.0)[:, None]
    cf_pad = jnp.pad(cf, ((0, npad - n), (0, 0)))

    x_pad, s1, s2 = pl.pallas_call(
        _x_stats_body,
        grid=(grid,),
        in_specs=[_smem_spec(), _row_spec(), _full_spec(_D, _D),
                  _full_spec(1, _D)],
        out_specs=[_row_spec(), _stat_spec(), _stat_spec()],
        out_shape=[jax.ShapeDtypeStruct((npad, _D), f32),
                   _stat_shape(grid), _stat_shape(grid)],
    )(numc_arr, cf_pad, W_cl, b_cl.reshape(1, _D))
    numcf = numc.astype(f32)
    m_c = jnp.sum(s1[:, 0, :], axis=0, keepdims=True) / numcf
    v_c = jnp.sum(s2[:, 0, :], axis=0, keepdims=True) / numcf - m_c * m_c

    cfn_pad = pl.pallas_call(
        _cfn_body,
        grid=(grid,),
        in_specs=[_smem_spec(), _row_spec(), _full_spec(1, _D),
                  _full_spec(1, _D), _full_spec(1, _D), _full_spec(1, _D)],
        out_specs=_row_spec(),
        out_shape=jax.ShapeDtypeStruct((npad, _D), f32),
    )(numc_arr, x_pad, m_c, v_c, g_cl.reshape(1, _D), be_cl.reshape(1, _D))
    cfn = cfn_pad[:n]

    key2 = key % 125000 + (b32 // 2) * 125000
    present2 = jnp.zeros((_KSP2,), jnp.int32).at[key2].max(1)
    excl2 = jnp.cumsum(present2) - present2
    group2 = excl2[key2]
    grp = jnp.zeros((n,), jnp.int32).at[pci].set(group2)
    vfarr = (jnp.arange(n) < numc).astype(f32)

    asum = jnp.zeros((n, _D), f32).at[grp].add(cfn * vfarr[:, None])
    acnt = jnp.zeros((n,), f32).at[grp].add(vfarr)
    amean = asum / jnp.maximum(acnt, 1.0)[:, None]

    pfc = cfn[pci]
    pfa = amean[group2]
    pf_pad = jnp.pad(points_feature, ((0, npad - n), (0, 0)))
    pfc_pad = jnp.pad(pfc, ((0, npad - n), (0, 0)))
    pfa_pad = jnp.pad(pfa, ((0, npad - n), (0, 0)))

    h_pad, hs1, hs2, hc = pl.pallas_call(
        _h_stats_body,
        grid=(grid,),
        in_specs=[_row_spec(), _row_spec(), _row_spec(),
                  _full_spec(_D, 3 * _D), _full_spec(1, _D)],
        out_specs=[_row_spec(), _stat_spec(), _stat_spec(), _stat_spec()],
        out_shape=[jax.ShapeDtypeStruct((npad, _D), f32),
                   _stat_shape(grid), _stat_shape(grid), _stat_shape(grid)],
    )(pf_pad, pfc_pad, pfa_pad, W_pf, b_pf.reshape(1, _D))
    cnt = jnp.sum(hc[:, 0, 0])
    csafe = jnp.maximum(cnt, 1.0)
    m_h = jnp.sum(hs1[:, 0, :], axis=0, keepdims=True) / csafe
    v_h = jnp.sum(hs2[:, 0, :], axis=0, keepdims=True) / csafe - m_h * m_h

    out_pad = pl.pallas_call(
        _out_body,
        grid=(grid,),
        in_specs=[_smem_spec(), _row_spec(), _row_spec(), _row_spec(),
                  _row_spec(), _full_spec(1, _D), _full_spec(1, _D),
                  _full_spec(1, _D), _full_spec(1, _D)],
        out_specs=_row_spec(),
        out_shape=jax.ShapeDtypeStruct((npad, _D), f32),
    )(cnt.reshape(1), h_pad, pfc_pad, pfa_pad, pf_pad, m_h, v_h,
      g_pf.reshape(1, _D), be_pf.reshape(1, _D))
    return out_pad[:n]


# speedup vs baseline: 2.3065x; 1.7414x over previous
import jax
import jax.numpy as jnp
from jax.experimental import pallas as pl
from jax.experimental.pallas import tpu as pltpu

_D = 128
_BLK = 1024
_KSP = 500000
_KSP2 = 250000


def _leaky(x):
    return jnp.where(x > 0, x, 0.1 * x)


def _dotT(a, w):
    return jax.lax.dot_general(a, w, (((1,), (1,)), ((), ())),
                               preferred_element_type=jnp.float32)


def _x_stats_body(numc_ref, cf_ref, w_ref, b_ref, x_ref, s1_ref, s2_ref):
    i = pl.program_id(0)
    x = _dotT(cf_ref[...], w_ref[...]) + b_ref[...]
    x_ref[...] = x
    rows = i * _BLK + jax.lax.broadcasted_iota(jnp.int32, (_BLK, 1), 0)
    vf = (rows < numc_ref[0]).astype(jnp.float32)
    xm = x * vf
    s1_ref[...] = jnp.sum(xm, axis=0).reshape(1, 1, _D)
    s2_ref[...] = jnp.sum(xm * x, axis=0).reshape(1, 1, _D)


def _cfn_body(numc_ref, x_ref, m_ref, v_ref, g_ref, be_ref, o_ref):
    i = pl.program_id(0)
    xn = _leaky(g_ref[...] * (x_ref[...] - m_ref[...])
                / jnp.sqrt(v_ref[...] + 1e-5) + be_ref[...])
    rows = i * _BLK + jax.lax.broadcasted_iota(jnp.int32, (_BLK, 1), 0)
    o_ref[...] = jnp.where(rows < numc_ref[0], xn, 0.0)


def _h_stats_body(pf_ref, pfc_ref, pfa_ref, w_ref, b_ref,
                  h_ref, s1_ref, s2_ref, c_ref):
    pf, pfc, pfa = pf_ref[...], pfc_ref[...], pfa_ref[...]
    w = w_ref[...]
    h = (_dotT(pf, w[:, 0:_D]) + _dotT(pfc, w[:, _D:2 * _D])
         + _dotT(pfa, w[:, 2 * _D:3 * _D])) + b_ref[...]
    h_ref[...] = h
    mf = (jnp.sum(pfc - pfa, axis=1, keepdims=True) > 0).astype(jnp.float32)
    hm = h * mf
    s1_ref[...] = jnp.sum(hm, axis=0).reshape(1, 1, _D)
    s2_ref[...] = jnp.sum(hm * h, axis=0).reshape(1, 1, _D)
    c_ref[...] = jnp.full((1, 1, _D), jnp.sum(mf), jnp.float32)


def _out_body(cnt_ref, h_ref, pfc_ref, pfa_ref, pf_ref, m_ref, v_ref,
              g_ref, be_ref, o_ref):
    hn = _leaky(g_ref[...] * (h_ref[...] - m_ref[...])
                / jnp.sqrt(v_ref[...] + 1e-5) + be_ref[...])
    mrow = jnp.sum(pfc_ref[...] - pfa_ref[...], axis=1, keepdims=True) > 0
    sel = jnp.logical_and(mrow, cnt_ref[0] > 1.0)
    o_ref[...] = jnp.where(sel, hn, pf_ref[...])


def _row_spec():
    return pl.BlockSpec((_BLK, _D), lambda i: (i, 0))


def _full_spec(r, c):
    return pl.BlockSpec((r, c), lambda i: (0, 0))


def _smem_spec():
    return pl.BlockSpec(memory_space=pltpu.SMEM)


def _stat_spec():
    return pl.BlockSpec((1, 1, _D), lambda i: (i, 0, 0))


def _stat_shape(grid):
    return jax.ShapeDtypeStruct((grid, 1, _D), jnp.float32)


def kernel(points_feature, points_center, batch_idx, points_cloud_timestamp,
           foreground_mask, W_pf, b_pf, g_pf, be_pf, W_cl, b_cl, g_cl, be_cl):
    n = points_feature.shape[0]
    npad = ((n + _BLK - 1) // _BLK) * _BLK
    grid = npad // _BLK
    f32 = jnp.float32

    b32 = batch_idx.astype(jnp.int32)
    xi = jnp.floor((points_center[:, 0] - 0.0) / 0.02).astype(jnp.int32)
    yi = jnp.floor((points_center[:, 1] - 0.0) / 0.02).astype(jnp.int32)
    zi = jnp.floor((points_center[:, 2] - 0.0) / 0.02).astype(jnp.int32)
    key = ((b32 * 50 + xi) * 50 + yi) * 50 + zi
    hist = jnp.zeros((_KSP,), jnp.int32).at[key].add(1)
    present = (hist > 0).astype(jnp.int32)
    excl = jnp.cumsum(present) - present
    tbl = jnp.stack([excl, hist], axis=1)
    gk = tbl[key]
    pci = gk[:, 0]
    numc = jnp.max(jnp.where(foreground_mask, pci, -1)) + 1
    numc_arr = numc.reshape(1)

    fmask = foreground_mask.astype(f32)
    pfw = points_feature * fmask[:, None]
    src = jnp.concatenate([pfw, fmask[:, None]], axis=1)
    acc = jnp.zeros((n, _D + 1), f32).at[pci].add(src)
    cf = acc[:, :_D] / jnp.maximum(acc[:, _D], 1.0)[:, None]
    cf_pad = jnp.pad(cf, ((0, npad - n), (0, 0)))

    x_pad, s1, s2 = pl.pallas_call(
        _x_stats_body,
        grid=(grid,),
        in_specs=[_smem_spec(), _row_spec(), _full_spec(_D, _D),
                  _full_spec(1, _D)],
        out_specs=[_row_spec(), _stat_spec(), _stat_spec()],
        out_shape=[jax.ShapeDtypeStruct((npad, _D), f32),
                   _stat_shape(grid), _stat_shape(grid)],
    )(numc_arr, cf_pad, W_cl, b_cl.reshape(1, _D))
    numcf = numc.astype(f32)
    m_c = jnp.sum(s1[:, 0, :], axis=0, keepdims=True) / numcf
    v_c = jnp.sum(s2[:, 0, :], axis=0, keepdims=True) / numcf - m_c * m_c

    cfn_pad = pl.pallas_call(
        _cfn_body,
        grid=(grid,),
        in_specs=[_smem_spec(), _row_spec(), _full_spec(1, _D),
                  _full_spec(1, _D), _full_spec(1, _D), _full_spec(1, _D)],
        out_specs=_row_spec(),
        out_shape=jax.ShapeDtypeStruct((npad, _D), f32),
    )(numc_arr, x_pad, m_c, v_c, g_cl.reshape(1, _D), be_cl.reshape(1, _D))
    cfn = cfn_pad[:n]

    kq = key + (1 - 2 * (b32 % 2)) * 125000
    gq = tbl[kq]
    q = gq[:, 0]
    presq = gq[:, 1] > 0
    pfc = cfn[pci]
    cfq = jnp.where(presq[:, None], cfn[q], 0.0)
    acnt_p = ((pci < numc).astype(f32)
              + jnp.logical_and(presq, q < numc).astype(f32))
    pfa = (pfc + cfq) / jnp.maximum(acnt_p, 1.0)[:, None]
    pf_pad = jnp.pad(points_feature, ((0, npad - n), (0, 0)))
    pfc_pad = jnp.pad(pfc, ((0, npad - n), (0, 0)))
    pfa_pad = jnp.pad(pfa, ((0, npad - n), (0, 0)))

    h_pad, hs1, hs2, hc = pl.pallas_call(
        _h_stats_body,
        grid=(grid,),
        in_specs=[_row_spec(), _row_spec(), _row_spec(),
                  _full_spec(_D, 3 * _D), _full_spec(1, _D)],
        out_specs=[_row_spec(), _stat_spec(), _stat_spec(), _stat_spec()],
        out_shape=[jax.ShapeDtypeStruct((npad, _D), f32),
                   _stat_shape(grid), _stat_shape(grid), _stat_shape(grid)],
    )(pf_pad, pfc_pad, pfa_pad, W_pf, b_pf.reshape(1, _D))
    cnt = jnp.sum(hc[:, 0, 0])
    csafe = jnp.maximum(cnt, 1.0)
    m_h = jnp.sum(hs1[:, 0, :], axis=0, keepdims=True) / csafe
    v_h = jnp.sum(hs2[:, 0, :], axis=0, keepdims=True) / csafe - m_h * m_h

    out_pad = pl.pallas_call(
        _out_body,
        grid=(grid,),
        in_specs=[_smem_spec(), _row_spec(), _row_spec(), _row_spec(),
                  _row_spec(), _full_spec(1, _D), _full_spec(1, _D),
                  _full_spec(1, _D), _full_spec(1, _D)],
        out_specs=_row_spec(),
        out_shape=jax.ShapeDtypeStruct((npad, _D), f32),
    )(cnt.reshape(1), h_pad, pfc_pad, pfa_pad, pf_pad, m_h, v_h,
      g_pf.reshape(1, _D), be_pf.reshape(1, _D))
    return out_pad[:n]
